# Initial kernel scaffold; baseline (speedup 1.0000x reference)
import jax
import jax.numpy as jnp
from jax.experimental import pallas as pl


def kernel(x, ln1_g, ln1_b, in_w, in_b, out_w, out_b, ln2_g, ln2_b, router_w, gate_w, gate_b, val_w, val_b, wo_w, wo_b):
    raise NotImplementedError("write your pallas kernel here")


# speedup vs baseline: 1.1071x; 1.1071x over previous
import jax
import jax.numpy as jnp
from jax.experimental import pallas as pl
from jax.experimental.pallas import tpu as pltpu

_INTERPRET = False

B, S, D, H, F, E = 1, 2048, 1024, 16, 4096, 8
HD = D // H
HALF = HD // 2
SBLK = 256
NSB = S // SBLK
QBLK = 512
EPAD = 128
FBLK = F // 2
NEG = -1e30


def _ln_f32(xb, g, b):
    m = jnp.mean(xb, axis=-1, keepdims=True)
    v = jnp.mean((xb - m) ** 2, axis=-1, keepdims=True)
    return (xb - m) / jnp.sqrt(v + 1e-5) * g + b


def _qkv_kernel(x_ref, inwT_ref, inb_ref, g_ref, b_ref, inv_ref, pswap_ref,
                q2_ref, k2_ref, v2_ref):
    s = pl.program_id(0)
    xb = x_ref[...]
    h = _ln_f32(xb, g_ref[...], b_ref[...])
    w = inwT_ref[...]
    qkv = jnp.dot(h.astype(jnp.bfloat16), w,
                  preferred_element_type=jnp.float32) + inb_ref[...]
    q, k, v = qkv[:, :D], qkv[:, D:2 * D], qkv[:, 2 * D:]
    pos = (s * SBLK
           + jax.lax.broadcasted_iota(jnp.int32, (SBLK, HALF), 0)).astype(jnp.float32)
    ang = pos * inv_ref[...]
    c32, s32 = jnp.cos(ang), jnp.sin(ang)
    c64 = jnp.concatenate([c32, c32], axis=1)
    s64 = jnp.concatenate([s32, s32], axis=1)
    cf = jnp.concatenate([c64] * H, axis=1)
    sf = jnp.concatenate([s64] * H, axis=1)
    psw = pswap_ref[...]
    qsw = jnp.dot(q.astype(jnp.bfloat16), psw, preferred_element_type=jnp.float32)
    ksw = jnp.dot(k.astype(jnp.bfloat16), psw, preferred_element_type=jnp.float32)
    rq = (q * cf + qsw * sf).astype(jnp.bfloat16)
    rk = (k * cf + ksw * sf).astype(jnp.bfloat16)
    q2_ref[...] = (jnp.dot(rq, w[:, :D], preferred_element_type=jnp.float32)
                   + inb_ref[:, :D]).astype(jnp.bfloat16)
    k2_ref[...] = (jnp.dot(rk, w[:, D:2 * D], preferred_element_type=jnp.float32)
                   + inb_ref[:, D:2 * D]).astype(jnp.bfloat16)
    v2_ref[...] = (jnp.dot(v.astype(jnp.bfloat16), w[:, 2 * D:],
                           preferred_element_type=jnp.float32)
                   + inb_ref[:, 2 * D:]).astype(jnp.bfloat16)


def _attn_kernel(q_ref, k_ref, v_ref, o_ref):
    qb = q_ref[...]
    kb = k_ref[...]
    vb = v_ref[...]
    outs = []
    for hh in range(2):
        q1 = qb[:, hh * HD:(hh + 1) * HD]
        k1 = kb[:, hh * HD:(hh + 1) * HD]
        v1 = vb[:, hh * HD:(hh + 1) * HD]
        sc = jax.lax.dot_general(q1, k1, (((1,), (1,)), ((), ())),
                                 preferred_element_type=jnp.float32) * 0.125
        m = jnp.max(sc, axis=1, keepdims=True)
        p = jnp.exp(sc - m)
        z = jnp.sum(p, axis=1, keepdims=True)
        pb = (p / z).astype(jnp.bfloat16)
        outs.append(jnp.dot(pb, v1, preferred_element_type=jnp.float32))
    o_ref[...] = jnp.concatenate(outs, axis=1).astype(jnp.bfloat16)


def _post_kernel(ctx_ref, x_ref, outwT_ref, outb_ref, g2_ref, b2_ref, rwT_ref,
                 x1_ref, h2_ref, w_ref):
    ctx = ctx_ref[...]
    attn = jnp.dot(ctx, outwT_ref[...],
                   preferred_element_type=jnp.float32) + outb_ref[...]
    x1 = x_ref[...] + attn
    x1_ref[...] = x1
    h2 = _ln_f32(x1, g2_ref[...], b2_ref[...])
    h2_ref[...] = h2.astype(jnp.bfloat16)
    logits = jnp.dot(h2, rwT_ref[...], preferred_element_type=jnp.float32)
    lane = jax.lax.broadcasted_iota(jnp.int32, (SBLK, EPAD), 1)
    l = jnp.where(lane < E, logits, NEG)
    m1 = jnp.max(l, axis=1, keepdims=True)
    i1 = jnp.min(jnp.where(l == m1, lane, EPAD), axis=1, keepdims=True)
    l2 = jnp.where(lane == i1, NEG, l)
    m2 = jnp.max(l2, axis=1, keepdims=True)
    i2 = jnp.min(jnp.where(l2 == m2, lane, EPAD), axis=1, keepdims=True)
    ex = jnp.where(lane < E, jnp.exp(l - m1), 0.0)
    zz = jnp.sum(ex, axis=1, keepdims=True)
    p = ex / zz
    w_ref[...] = jnp.where((lane == i1) | (lane == i2), p, 0.0)


def _moe_kernel(h2_ref, x1_ref, w_ref, gw_ref, vw_ref, ow_ref,
                gb_ref, vb_ref, ob_ref, out_ref):
    e = pl.program_id(1)
    f = pl.program_id(2)
    si = pl.program_id(3)
    xb = h2_ref[...]
    g = jax.lax.dot_general(xb, gw_ref[0], (((1,), (1,)), ((), ())),
                            preferred_element_type=jnp.float32) + gb_ref[0, 0]
    v = jax.lax.dot_general(xb, vw_ref[0], (((1,), (1,)), ((), ())),
                            preferred_element_type=jnp.float32) + vb_ref[0, 0]
    hh = (v * (g * jax.nn.sigmoid(g))).astype(jnp.bfloat16)
    o = jax.lax.dot_general(hh, ow_ref[0], (((1,), (1,)), ((), ())),
                            preferred_element_type=jnp.float32)
    lane = jax.lax.broadcasted_iota(jnp.int32, (SBLK, EPAD), 1)
    wsel = jnp.sum(jnp.where(lane == e, w_ref[...], 0.0), axis=1, keepdims=True)
    ob = ob_ref[0, 0][None, :] * (f == 0).astype(jnp.float32)
    contrib = (o + ob) * wsel
    first = (e == 0) & (f == 0)

    @pl.when(first)
    def _():
        out_ref[pl.ds(si * SBLK, SBLK), :] = x1_ref[...] + contrib

    @pl.when(jnp.logical_not(first))
    def _():
        out_ref[pl.ds(si * SBLK, SBLK), :] += contrib


def _cparams(sem):
    return pltpu.CompilerParams(dimension_semantics=sem)


def kernel(x, ln1_g, ln1_b, in_w, in_b, out_w, out_b, ln2_g, ln2_b,
           router_w, gate_w, gate_b, val_w, val_b, wo_w, wo_b):
    f32, bf16 = jnp.float32, jnp.bfloat16
    x2 = x.reshape(S, D)
    inwT = in_w.T.astype(bf16)
    inb = in_b.reshape(1, 3 * D)
    g1 = ln1_g.reshape(1, D)
    b1 = ln1_b.reshape(1, D)
    inv = (1.0 / (10000.0 ** (jnp.arange(HALF, dtype=f32) / HALF))).reshape(1, HALF)
    eye = jnp.eye(HALF, dtype=f32)
    zer = jnp.zeros((HALF, HALF), f32)
    p64 = jnp.concatenate([
        jnp.concatenate([zer, eye], axis=1),
        jnp.concatenate([-eye, zer], axis=1)], axis=0)
    pswap = jnp.kron(jnp.eye(H, dtype=f32), p64).astype(bf16)

    q2, k2, v2 = pl.pallas_call(
        _qkv_kernel,
        grid=(NSB,),
        in_specs=[
            pl.BlockSpec((SBLK, D), lambda s: (s, 0)),
            pl.BlockSpec((D, 3 * D), lambda s: (0, 0)),
            pl.BlockSpec((1, 3 * D), lambda s: (0, 0)),
            pl.BlockSpec((1, D), lambda s: (0, 0)),
            pl.BlockSpec((1, D), lambda s: (0, 0)),
            pl.BlockSpec((1, HALF), lambda s: (0, 0)),
            pl.BlockSpec((D, D), lambda s: (0, 0)),
        ],
        out_specs=[pl.BlockSpec((SBLK, D), lambda s: (s, 0))] * 3,
        out_shape=[jax.ShapeDtypeStruct((S, D), bf16)] * 3,
        compiler_params=_cparams(("parallel",)),
        interpret=_INTERPRET,
    )(x2, inwT, inb, g1, b1, inv, pswap)

    ctx = pl.pallas_call(
        _attn_kernel,
        grid=(H // 2, S // QBLK),
        in_specs=[
            pl.BlockSpec((QBLK, 2 * HD), lambda hp, sq: (sq, hp)),
            pl.BlockSpec((S, 2 * HD), lambda hp, sq: (0, hp)),
            pl.BlockSpec((S, 2 * HD), lambda hp, sq: (0, hp)),
        ],
        out_specs=pl.BlockSpec((QBLK, 2 * HD), lambda hp, sq: (sq, hp)),
        out_shape=jax.ShapeDtypeStruct((S, D), bf16),
        compiler_params=_cparams(("parallel", "parallel")),
        interpret=_INTERPRET,
    )(q2, k2, v2)

    outwT = out_w.T.astype(bf16)
    outb = out_b.reshape(1, D)
    g2 = ln2_g.reshape(1, D)
    b2 = ln2_b.reshape(1, D)
    rwT = jnp.zeros((D, EPAD), f32).at[:, :E].set(router_w.T)

    x1, h2, w = pl.pallas_call(
        _post_kernel,
        grid=(NSB,),
        in_specs=[
            pl.BlockSpec((SBLK, D), lambda s: (s, 0)),
            pl.BlockSpec((SBLK, D), lambda s: (s, 0)),
            pl.BlockSpec((D, D), lambda s: (0, 0)),
            pl.BlockSpec((1, D), lambda s: (0, 0)),
            pl.BlockSpec((1, D), lambda s: (0, 0)),
            pl.BlockSpec((1, D), lambda s: (0, 0)),
            pl.BlockSpec((D, EPAD), lambda s: (0, 0)),
        ],
        out_specs=[
            pl.BlockSpec((SBLK, D), lambda s: (s, 0)),
            pl.BlockSpec((SBLK, D), lambda s: (s, 0)),
            pl.BlockSpec((SBLK, EPAD), lambda s: (s, 0)),
        ],
        out_shape=[
            jax.ShapeDtypeStruct((S, D), f32),
            jax.ShapeDtypeStruct((S, D), bf16),
            jax.ShapeDtypeStruct((S, EPAD), f32),
        ],
        compiler_params=_cparams(("parallel",)),
        interpret=_INTERPRET,
    )(ctx, x2, outwT, outb, g2, b2, rwT)

    gwb = gate_w.astype(bf16)
    vwb = val_w.astype(bf16)
    owb = wo_w.astype(bf16)
    gb3 = gate_b.reshape(E * 2, 1, FBLK)
    vb3 = val_b.reshape(E * 2, 1, FBLK)
    ob3 = wo_b.reshape(E, 1, D)

    out = pl.pallas_call(
        _moe_kernel,
        grid=(2, E, 2, NSB // 2),
        in_specs=[
            pl.BlockSpec((SBLK, D), lambda s2, e, f, si: (s2 * (NSB // 2) + si, 0)),
            pl.BlockSpec((SBLK, D), lambda s2, e, f, si: (s2 * (NSB // 2) + si, 0)),
            pl.BlockSpec((SBLK, EPAD), lambda s2, e, f, si: (s2 * (NSB // 2) + si, 0)),
            pl.BlockSpec((1, FBLK, D), lambda s2, e, f, si: (e, f, 0)),
            pl.BlockSpec((1, FBLK, D), lambda s2, e, f, si: (e, f, 0)),
            pl.BlockSpec((1, D, FBLK), lambda s2, e, f, si: (e, 0, f)),
            pl.BlockSpec((1, 1, FBLK), lambda s2, e, f, si: (e * 2 + f, 0, 0)),
            pl.BlockSpec((1, 1, FBLK), lambda s2, e, f, si: (e * 2 + f, 0, 0)),
            pl.BlockSpec((1, 1, D), lambda s2, e, f, si: (e, 0, 0)),
        ],
        out_specs=pl.BlockSpec((S // 2, D), lambda s2, e, f, si: (s2, 0)),
        out_shape=jax.ShapeDtypeStruct((S, D), f32),
        compiler_params=_cparams(("parallel", "arbitrary", "arbitrary", "arbitrary")),
        interpret=_INTERPRET,
    )(h2, x1, w, gwb, vwb, owb, gb3, vb3, ob3)

    return out.reshape(B, S, D)
